# baseline (speedup 1.0000x reference)
import functools

import jax
import jax.numpy as jnp
from jax.experimental import pallas as pl
from jax.experimental.pallas import tpu as pltpu

_K = 1024
_D = 256
_B = 16
_HW = 1024
_N = 16 * 32 * 32
_R = 512
_SPLIT = _HW // _R
_NB = _N // _R
_BETA = 0.25


def _vq_body(z_ref, w_ref,
             oh_ref, idx_ref, sc_ref, zq_ref, loss_ref, perp_ref, md_ref,
             cnt_ref, dsum_ref, lsum_ref):
    i = pl.program_id(0)
    zt = z_ref[...]
    w = w_ref[...]

    zsq = jnp.sum(zt * zt, axis=1, keepdims=True)
    wsq = jnp.sum(w * w, axis=1)
    mm = jax.lax.dot_general(zt, w, (((1,), (1,)), ((), ())))
    d = (zsq + wsq[None, :]) - 2.0 * mm

    m = jnp.min(d, axis=1, keepdims=True)
    ids = jax.lax.broadcasted_iota(jnp.int32, d.shape, 1)
    idx = jnp.min(jnp.where(d == m, ids, _K), axis=1)
    oh = (ids == idx[:, None]).astype(jnp.float32)

    oh_ref[...] = oh
    idx_ref[...] = idx
    sc_ref[...] = jnp.exp(-m[:, 0] / 10.0)
    zq = jax.lax.dot_general(oh, w, (((1,), (0,)), ((), ())))
    zq_ref[0, :, pl.ds((i % _SPLIT) * _R, _R)] = jnp.transpose(zq, (1, 0))

    pc = jnp.sum(oh, axis=0, keepdims=True)
    ds = jnp.sum(d)
    ls = jnp.sum((zq - zt) ** 2)

    @pl.when(i == 0)
    def _init():
        cnt_ref[...] = pc
        dsum_ref[0] = ds
        lsum_ref[0] = ls

    @pl.when(i > 0)
    def _acc():
        cnt_ref[...] = cnt_ref[...] + pc
        dsum_ref[0] = dsum_ref[0] + ds
        lsum_ref[0] = lsum_ref[0] + ls

    mean_l = lsum_ref[0] / jnp.float32(_N * _D)
    loss_ref[...] = jnp.reshape(mean_l + _BETA * mean_l, (1, 1))
    md_ref[...] = jnp.reshape(dsum_ref[0] / jnp.float32(_N * _K), (1, 1))
    e_mean = cnt_ref[...] * jnp.float32(1.0 / _N)
    ent = jnp.sum(e_mean * jnp.log(e_mean + 1e-10))
    perp_ref[...] = jnp.reshape(jnp.exp(-ent), (1, 1))


@functools.partial(jax.jit)
def _vq(zf, W):
    grid = (_NB,)
    out_shapes = [
        jax.ShapeDtypeStruct((_N, _K), jnp.float32),
        jax.ShapeDtypeStruct((_N,), jnp.int32),
        jax.ShapeDtypeStruct((_N,), jnp.float32),
        jax.ShapeDtypeStruct((_B, _D, _HW), jnp.float32),
        jax.ShapeDtypeStruct((1, 1), jnp.float32),
        jax.ShapeDtypeStruct((1, 1), jnp.float32),
        jax.ShapeDtypeStruct((1, 1), jnp.float32),
    ]
    out_specs = [
        pl.BlockSpec((_R, _K), lambda i: (i, 0)),
        pl.BlockSpec((_R,), lambda i: (i,)),
        pl.BlockSpec((_R,), lambda i: (i,)),
        pl.BlockSpec((1, _D, _HW), lambda i: (i // _SPLIT, 0, 0)),
        pl.BlockSpec((1, 1), lambda i: (0, 0)),
        pl.BlockSpec((1, 1), lambda i: (0, 0)),
        pl.BlockSpec((1, 1), lambda i: (0, 0)),
    ]
    in_specs = [
        pl.BlockSpec((_R, _D), lambda i: (i, 0)),
        pl.BlockSpec((_K, _D), lambda i: (0, 0)),
    ]
    return pl.pallas_call(
        _vq_body,
        grid=grid,
        in_specs=in_specs,
        out_specs=out_specs,
        out_shape=out_shapes,
        scratch_shapes=[
            pltpu.VMEM((1, _K), jnp.float32),
            pltpu.SMEM((1,), jnp.float32),
            pltpu.SMEM((1,), jnp.float32),
        ],
    )(zf, W)


def kernel(z, W):
    B, C, H, Wd = z.shape
    zf = jnp.transpose(z, (0, 2, 3, 1)).reshape(-1, C)
    oh, idx, sc, zq, loss, perp, md = _vq(zf, W)
    z_q = zq.reshape(B, C, H, Wd)
    return (z_q,
            loss[0, 0],
            perp[0, 0],
            oh,
            idx.reshape(-1, 1),
            sc.reshape(-1, 1),
            md[0, 0])


# speedup vs baseline: 1.2449x; 1.2449x over previous
import functools

import jax
import jax.numpy as jnp
from jax.experimental import pallas as pl
from jax.experimental.pallas import tpu as pltpu

_K = 1024
_D = 256
_N = 16 * 32 * 32
_R = 512
_NB = _N // _R
_BETA = 0.25


def _vq_body(z_ref, w_ref,
             oh_ref, idx_ref, sc_ref, zq_ref, loss_ref, perp_ref, md_ref,
             cnt_ref, dsum_ref, lsum_ref):
    i = pl.program_id(0)
    zt = z_ref[...]
    w = w_ref[...]

    zsq = jnp.sum(zt * zt, axis=1, keepdims=True)
    wsq = jnp.sum(w * w, axis=1)
    mm = jax.lax.dot_general(zt, w, (((1,), (1,)), ((), ())))
    d = (zsq + wsq[None, :]) - 2.0 * mm

    m = jnp.min(d, axis=1, keepdims=True)
    ids = jax.lax.broadcasted_iota(jnp.int32, d.shape, 1)
    idx = jnp.min(jnp.where(d == m, ids, _K), axis=1)
    oh = (ids == idx[:, None]).astype(jnp.float32)

    oh_ref[...] = oh
    idx_ref[...] = idx
    sc_ref[...] = jnp.exp(-m[:, 0] / 10.0)
    zq = jax.lax.dot_general(oh, w, (((1,), (0,)), ((), ())))
    zq_ref[...] = zq

    pc = jnp.sum(oh, axis=0, keepdims=True)
    ds = jnp.sum(d)
    ls = jnp.sum((zq - zt) ** 2)

    @pl.when(i == 0)
    def _init():
        cnt_ref[...] = pc
        dsum_ref[0] = ds
        lsum_ref[0] = ls

    @pl.when(i > 0)
    def _acc():
        cnt_ref[...] = cnt_ref[...] + pc
        dsum_ref[0] = dsum_ref[0] + ds
        lsum_ref[0] = lsum_ref[0] + ls

    mean_l = lsum_ref[0] / jnp.float32(_N * _D)
    loss_ref[...] = jnp.reshape(mean_l + _BETA * mean_l, (1, 1))
    md_ref[...] = jnp.reshape(dsum_ref[0] / jnp.float32(_N * _K), (1, 1))
    e_mean = cnt_ref[...] * jnp.float32(1.0 / _N)
    ent = jnp.sum(e_mean * jnp.log(e_mean + 1e-10))
    perp_ref[...] = jnp.reshape(jnp.exp(-ent), (1, 1))


@functools.partial(jax.jit)
def _vq(zf, W):
    grid = (_NB,)
    out_shapes = [
        jax.ShapeDtypeStruct((_N, _K), jnp.float32),
        jax.ShapeDtypeStruct((_N,), jnp.int32),
        jax.ShapeDtypeStruct((_N,), jnp.float32),
        jax.ShapeDtypeStruct((_N, _D), jnp.float32),
        jax.ShapeDtypeStruct((1, 1), jnp.float32),
        jax.ShapeDtypeStruct((1, 1), jnp.float32),
        jax.ShapeDtypeStruct((1, 1), jnp.float32),
    ]
    out_specs = [
        pl.BlockSpec((_R, _K), lambda i: (i, 0)),
        pl.BlockSpec((_R,), lambda i: (i,)),
        pl.BlockSpec((_R,), lambda i: (i,)),
        pl.BlockSpec((_R, _D), lambda i: (i, 0)),
        pl.BlockSpec((1, 1), lambda i: (0, 0)),
        pl.BlockSpec((1, 1), lambda i: (0, 0)),
        pl.BlockSpec((1, 1), lambda i: (0, 0)),
    ]
    in_specs = [
        pl.BlockSpec((_R, _D), lambda i: (i, 0)),
        pl.BlockSpec((_K, _D), lambda i: (0, 0)),
    ]
    return pl.pallas_call(
        _vq_body,
        grid=grid,
        in_specs=in_specs,
        out_specs=out_specs,
        out_shape=out_shapes,
        scratch_shapes=[
            pltpu.VMEM((1, _K), jnp.float32),
            pltpu.SMEM((1,), jnp.float32),
            pltpu.SMEM((1,), jnp.float32),
        ],
    )(zf, W)


def kernel(z, W):
    B, C, H, Wd = z.shape
    zf = jnp.transpose(z, (0, 2, 3, 1)).reshape(-1, C)
    oh, idx, sc, zq, loss, perp, md = _vq(zf, W)
    z_q = zq.reshape(B, H, Wd, C).transpose(0, 3, 1, 2)
    return (z_q,
            loss[0, 0],
            perp[0, 0],
            oh,
            idx.reshape(-1, 1),
            sc.reshape(-1, 1),
            md[0, 0])
